# baseline (speedup 1.0000x reference)
import jax
import jax.numpy as jnp
from jax import lax
from jax.experimental import pallas as pl
from jax.experimental.pallas import tpu as pltpu

_C = 1000
_EPS = 1e-05


def _tc_body(n_total, t_ref, x_ref, out_ref, cnt_ref, seg_ref):
    i = pl.program_id(0)
    n = pl.num_programs(0)
    x = x_ref[...]
    t = t_ref[0, 0, :]
    m = jnp.max(x, axis=1, keepdims=True)
    e = jnp.exp(x - m)
    s = jnp.sum(e, axis=1)
    lse = m[:, 0] + jnp.log(s)
    cols = lax.broadcasted_iota(jnp.int32, x.shape, 1)
    maskf = (cols == t[:, None]).astype(jnp.float32)
    picked = jnp.sum(maskf * x, axis=1)
    loss = lse - picked
    cnt_part = jnp.sum(maskf, axis=0)
    seg_part = jnp.sum(maskf * loss[:, None], axis=0)

    @pl.when(i == 0)
    def _():
        cnt_ref[0, :] = cnt_part
        seg_ref[0, :] = seg_part

    @pl.when(i > 0)
    def _():
        cnt_ref[0, :] = cnt_ref[0, :] + cnt_part
        seg_ref[0, :] = seg_ref[0, :] + seg_part

    @pl.when(i == n - 1)
    def _():
        cnt = cnt_ref[0, :]
        seg = seg_ref[0, :]
        w = 1.0 / (cnt + _EPS)
        total_w = jnp.sum(w)
        out_ref[0, 0] = jnp.sum(seg * w) * (_C / total_w) / n_total


def kernel(inputs, targets):
    n_total, c = inputs.shape
    block = 4096
    grid = n_total // block
    t3 = targets.astype(jnp.int32).reshape(grid, 1, block)
    import functools
    body = functools.partial(_tc_body, float(n_total))
    out = pl.pallas_call(
        body,
        grid=(grid,),
        in_specs=[
            pl.BlockSpec((1, 1, block), lambda i: (i, 0, 0)),
            pl.BlockSpec((block, c), lambda i: (i, 0)),
        ],
        out_specs=pl.BlockSpec((1, 1), lambda i: (0, 0),
                               memory_space=pltpu.SMEM),
        out_shape=jax.ShapeDtypeStruct((1, 1), jnp.float32),
        scratch_shapes=[
            pltpu.VMEM((1, c), jnp.float32),
            pltpu.VMEM((1, c), jnp.float32),
        ],
    )(t3, inputs)
    return out[0, 0]


# speedup vs baseline: 1.0466x; 1.0466x over previous
import jax
import jax.numpy as jnp
from jax import lax
from jax.experimental import pallas as pl
from jax.experimental.pallas import tpu as pltpu

_C = 1000
_EPS = 1e-05


def _tc_body(n_total, t_ref, x_ref, out_ref, acc_ref):
    i = pl.program_id(0)
    n = pl.num_programs(0)
    x = x_ref[...]
    t = t_ref[0, 0, :]
    m = jnp.max(x, axis=1, keepdims=True)
    e = jnp.exp(x - m)
    s = jnp.sum(e, axis=1)
    lse = m[:, 0] + jnp.log(s)
    cols = lax.broadcasted_iota(jnp.int32, x.shape, 1)
    maskf = (cols == t[:, None]).astype(jnp.float32)
    picked = jnp.sum(maskf * x, axis=1)
    loss = lse - picked
    lhs = jnp.concatenate(
        [loss[None, :], jnp.ones((1, loss.shape[0]), jnp.float32)], axis=0)
    segcnt = jax.lax.dot_general(
        lhs, maskf, (((1,), (0,)), ((), ())),
        preferred_element_type=jnp.float32)

    @pl.when(i == 0)
    def _():
        acc_ref[...] = segcnt

    @pl.when(i > 0)
    def _():
        acc_ref[...] = acc_ref[...] + segcnt

    @pl.when(i == n - 1)
    def _():
        cnt = acc_ref[1, :]
        seg = acc_ref[0, :]
        w = 1.0 / (cnt + _EPS)
        total_w = jnp.sum(w)
        out_ref[0, 0] = jnp.sum(seg * w) * (_C / total_w) / n_total


def kernel(inputs, targets):
    n_total, c = inputs.shape
    block = 2048
    grid = n_total // block
    t3 = targets.astype(jnp.int32).reshape(grid, 1, block)
    import functools
    body = functools.partial(_tc_body, float(n_total))
    out = pl.pallas_call(
        body,
        grid=(grid,),
        in_specs=[
            pl.BlockSpec((1, 1, block), lambda i: (i, 0, 0)),
            pl.BlockSpec((block, c), lambda i: (i, 0)),
        ],
        out_specs=pl.BlockSpec((1, 1), lambda i: (0, 0),
                               memory_space=pltpu.SMEM),
        out_shape=jax.ShapeDtypeStruct((1, 1), jnp.float32),
        scratch_shapes=[
            pltpu.VMEM((2, c), jnp.float32),
        ],
    )(t3, inputs)
    return out[0, 0]
